# Initial kernel scaffold; baseline (speedup 1.0000x reference)
import jax
import jax.numpy as jnp
from jax.experimental import pallas as pl


def kernel(x, edge_index, W1, b1, W2, b2):
    raise NotImplementedError("write your pallas kernel here")


# speedup vs baseline: 7.4621x; 7.4621x over previous
import jax
import jax.numpy as jnp
from jax import lax
from jax.experimental import pallas as pl
from jax.experimental.pallas import tpu as pltpu
from jax.experimental.pallas import tpu_sc as plsc

N = 10000
E = 160000
D = 256
DQ = 64
NP = 10240
ECH = 128
EP = 163840
DUMP = N
NSUB = 16
RPT = NP // NSUB
NCHT = EP // ECH // NSUB
NCHD = EP // ECH // (2 * NSUB)
NBUF = 4
BN = 256

_mesh = plsc.VectorSubcoreMesh(
    core_axis_name="c", subcore_axis_name="s", num_cores=2, num_subcores=NSUB
)


def _deg_body(dst2d, ones16, z16, degp, dst_buf, ones_v, acc):
    cid = lax.axis_index("c")
    sid = lax.axis_index("s")
    wid = cid * NSUB + sid
    pltpu.sync_copy(z16, acc.at[pl.ds(sid * RPT, RPT)])
    pltpu.sync_copy(ones16, ones_v)
    pltpu.sync_copy(dst2d.at[pl.ds(wid * NCHD, NCHD)], dst_buf)
    plsc.subcore_barrier()

    @pl.loop(0, NCHD)
    def _(i):
        pltpu.sync_copy(ones_v, acc.at[dst_buf.at[i]], add=True)

    plsc.subcore_barrier()
    pltpu.sync_copy(
        acc.at[pl.ds(sid * RPT, RPT)], degp.at[cid, pl.ds(sid * RPT, RPT)]
    )


_deg_call = pl.kernel(
    _deg_body,
    out_type=jax.ShapeDtypeStruct((2, NP, 16), jnp.float32),
    mesh=_mesh,
    scratch_types=[
        pltpu.VMEM((NCHD, ECH), jnp.int32),
        pltpu.VMEM((ECH, 16), jnp.float32),
        pltpu.VMEM_SHARED((NP, 16), jnp.float32),
    ],
    compiler_params=pltpu.CompilerParams(use_tc_tiling_on_sc=False),
)


def _seg_body(g0, g1, g2, g3, src2d, dst2d, zrows, s0, s1, s2, s3,
              src_buf, dst_buf, data, acc, gsem):
    cid = lax.axis_index("c")
    sid = lax.axis_index("s")
    base = sid * NCHT
    pltpu.sync_copy(src2d.at[pl.ds(base, NCHT)], src_buf)
    pltpu.sync_copy(dst2d.at[pl.ds(base, NCHT)], dst_buf)

    def run(g, s):
        pltpu.sync_copy(zrows, acc.at[pl.ds(sid * RPT, RPT)])
        plsc.subcore_barrier()

        for b in range(NBUF):
            pltpu.async_copy(g.at[src_buf.at[b]], data.at[b], gsem)

        @pl.loop(0, NCHT - NBUF)
        def _(i):
            slot = lax.rem(i, NBUF)
            pltpu.make_async_copy(
                g.at[src_buf.at[i]], data.at[slot], gsem
            ).wait()
            pltpu.sync_copy(data.at[slot], acc.at[dst_buf.at[i]], add=True)
            pltpu.async_copy(g.at[src_buf.at[i + NBUF]], data.at[slot], gsem)

        @pl.loop(NCHT - NBUF, NCHT)
        def _(i):
            slot = lax.rem(i, NBUF)
            pltpu.make_async_copy(
                g.at[src_buf.at[i]], data.at[slot], gsem
            ).wait()
            pltpu.sync_copy(data.at[slot], acc.at[dst_buf.at[i]], add=True)

        plsc.subcore_barrier()
        pltpu.sync_copy(
            acc.at[pl.ds(sid * RPT, RPT)], s.at[pl.ds(sid * RPT, RPT)]
        )

    for r, (ga, gb, sa, sb) in enumerate(((g0, g1, s0, s1),
                                          (g2, g3, s2, s3))):
        @pl.when(cid == 0)
        def _(ga=ga, sa=sa):
            run(ga, sa)

        @pl.when(cid == 1)
        def _(gb=gb, sb=sb):
            run(gb, sb)


_seg_call = pl.kernel(
    _seg_body,
    out_type=(jax.ShapeDtypeStruct((NP, DQ), jnp.float32),) * 4,
    mesh=_mesh,
    scratch_types=[
        pltpu.VMEM((NCHT, ECH), jnp.int32),
        pltpu.VMEM((NCHT, ECH), jnp.int32),
        pltpu.VMEM((NBUF, ECH, DQ), jnp.float32),
        pltpu.VMEM_SHARED((NP, DQ), jnp.float32),
        pltpu.SemaphoreType.DMA,
    ],
    compiler_params=pltpu.CompilerParams(use_tc_tiling_on_sc=False),
)


def _dinv_of(degp_ref):
    deg = 1.0 + degp_ref[0][:, 0:1] + degp_ref[1][:, 0:1]
    return lax.rsqrt(deg)


def _tc1_body(x_ref, w_ref, degp_ref, *g_refs):
    dinv = _dinv_of(degp_ref)
    h = jnp.dot(x_ref[...], w_ref[...], preferred_element_type=jnp.float32)
    g = h * dinv
    for q, g_ref in enumerate(g_refs):
        g_ref[...] = g[:, q * DQ:(q + 1) * DQ]


def _tc2_body(s0, s1, s2, s3, g0, g1, g2, g3, degp_ref, b_ref, w_ref,
              *g2_refs):
    dinv = _dinv_of(degp_ref)
    s = jnp.concatenate([s0[...], s1[...], s2[...], s3[...]], axis=1)
    g = jnp.concatenate([g0[...], g1[...], g2[...], g3[...]], axis=1)
    a = jnp.maximum((s + g) * dinv + b_ref[...], 0.0)
    h = jnp.dot(a, w_ref[...], preferred_element_type=jnp.float32)
    gg = h * dinv
    for q, g_ref in enumerate(g2_refs):
        g_ref[...] = gg[:, q * DQ:(q + 1) * DQ]


def _tc3_body(s0, s1, s2, s3, g0, g1, g2, g3, degp_ref, b_ref, x_ref,
              out_ref):
    dinv = _dinv_of(degp_ref)
    s = jnp.concatenate([s0[...], s1[...], s2[...], s3[...]], axis=1)
    g = jnp.concatenate([g0[...], g1[...], g2[...], g3[...]], axis=1)
    out_ref[...] = (s + g) * dinv + b_ref[...] + x_ref[...]


_q_spec = pl.BlockSpec((BN, DQ), lambda i: (i, 0))
_full_spec = pl.BlockSpec((BN, D), lambda i: (i, 0))
_w_spec = pl.BlockSpec((D, D), lambda i: (0, 0))
_degp_spec = pl.BlockSpec((2, BN, 16), lambda i: (0, i, 0))
_b_spec = pl.BlockSpec((1, D), lambda i: (0, 0))

_tc1_call = pl.pallas_call(
    _tc1_body,
    grid=(NP // BN,),
    in_specs=[_full_spec, _w_spec, _degp_spec],
    out_specs=[_q_spec] * 4,
    out_shape=[jax.ShapeDtypeStruct((NP, DQ), jnp.float32)] * 4,
)

_tc2_call = pl.pallas_call(
    _tc2_body,
    grid=(NP // BN,),
    in_specs=[_q_spec] * 8 + [_degp_spec, _b_spec, _w_spec],
    out_specs=[_q_spec] * 4,
    out_shape=[jax.ShapeDtypeStruct((NP, DQ), jnp.float32)] * 4,
)

_tc3_call = pl.pallas_call(
    _tc3_body,
    grid=(NP // BN,),
    in_specs=[_q_spec] * 8 + [_degp_spec, _b_spec, _full_spec],
    out_specs=_full_spec,
    out_shape=jax.ShapeDtypeStruct((NP, D), jnp.float32),
)


def kernel(x, edge_index, W1, b1, W2, b2):
    src = edge_index[0]
    dst = edge_index[1]
    pad = jnp.full((EP - E,), DUMP, dtype=jnp.int32)
    src2d = jnp.concatenate([src, pad]).reshape(EP // ECH, ECH)
    dst2d = jnp.concatenate([dst, pad]).reshape(EP // ECH, ECH)
    xp = jnp.zeros((NP, D), jnp.float32).at[:N].set(x)
    w1t = W1.T
    w2t = W2.T
    zrows = jnp.zeros((RPT, DQ), jnp.float32)
    z16 = jnp.zeros((RPT, 16), jnp.float32)
    ones16 = jnp.zeros((ECH, 16), jnp.float32).at[:, 0].set(1.0)

    degp = _deg_call(dst2d, ones16, z16)
    gs = _tc1_call(xp, w1t, degp)
    ss = _seg_call(*gs, src2d, dst2d, zrows)
    g2s = _tc2_call(*ss, *gs, degp, b1.reshape(1, D), w2t)
    s2s = _seg_call(*g2s, src2d, dst2d, zrows)
    out = _tc3_call(*s2s, *g2s, degp, b2.reshape(1, D), xp)
    return out[:N]


# speedup vs baseline: 7.4938x; 1.0043x over previous
import jax
import jax.numpy as jnp
from jax import lax
from jax.experimental import pallas as pl
from jax.experimental.pallas import tpu as pltpu
from jax.experimental.pallas import tpu_sc as plsc

N = 10000
E = 160000
D = 256
DQ = 64
NP = 10240
ECH = 128
EP = 163840
DUMP = N
NSUB = 16
RPT = NP // NSUB
NCHT = EP // ECH // NSUB
NCHD = EP // ECH // (2 * NSUB)
NBUF = 6
BN = 256

_mesh = plsc.VectorSubcoreMesh(
    core_axis_name="c", subcore_axis_name="s", num_cores=2, num_subcores=NSUB
)


def _deg_body(dst2d, ones16, z16, degp, dst_buf, ones_v, acc):
    cid = lax.axis_index("c")
    sid = lax.axis_index("s")
    wid = cid * NSUB + sid
    pltpu.sync_copy(z16, acc.at[pl.ds(sid * RPT, RPT)])
    pltpu.sync_copy(ones16, ones_v)
    pltpu.sync_copy(dst2d.at[pl.ds(wid * NCHD, NCHD)], dst_buf)
    plsc.subcore_barrier()

    @pl.loop(0, NCHD)
    def _(i):
        pltpu.sync_copy(ones_v, acc.at[dst_buf.at[i]], add=True)

    plsc.subcore_barrier()
    pltpu.sync_copy(
        acc.at[pl.ds(sid * RPT, RPT)], degp.at[cid, pl.ds(sid * RPT, RPT)]
    )


_deg_call = pl.kernel(
    _deg_body,
    out_type=jax.ShapeDtypeStruct((2, NP, 16), jnp.float32),
    mesh=_mesh,
    scratch_types=[
        pltpu.VMEM((NCHD, ECH), jnp.int32),
        pltpu.VMEM((ECH, 16), jnp.float32),
        pltpu.VMEM_SHARED((NP, 16), jnp.float32),
    ],
    compiler_params=pltpu.CompilerParams(use_tc_tiling_on_sc=False),
)


def _seg_body(g0, g1, g2, g3, src2d, dst2d, zrows, s0, s1, s2, s3,
              src_buf, dst_buf, data, acc, gsem, ssem):
    cid = lax.axis_index("c")
    sid = lax.axis_index("s")
    base = sid * NCHT
    pltpu.sync_copy(src2d.at[pl.ds(base, NCHT)], src_buf)
    pltpu.sync_copy(dst2d.at[pl.ds(base, NCHT)], dst_buf)

    def run(g, s):
        pltpu.sync_copy(zrows, acc.at[pl.ds(sid * RPT, RPT)])
        plsc.subcore_barrier()

        def gath(i):
            pltpu.async_copy(
                g.at[src_buf.at[i]], data.at[lax.rem(i, NBUF)], gsem
            )

        def wait_gath(i):
            pltpu.make_async_copy(
                g.at[src_buf.at[i]], data.at[lax.rem(i, NBUF)], gsem
            ).wait()

        def scat(i):
            pltpu.async_copy(
                data.at[lax.rem(i, NBUF)], acc.at[dst_buf.at[i]], ssem,
                add=True,
            )

        def wait_scat(i):
            pltpu.make_async_copy(
                data.at[lax.rem(i, NBUF)], acc.at[dst_buf.at[i]], ssem
            ).wait()

        for b in range(NBUF):
            gath(b)

        @pl.loop(0, NCHT)
        def _(i):
            wait_gath(i)
            scat(i)

            @pl.when(i >= 2)
            def _():
                wait_scat(i - 2)

                @pl.when(i + NBUF - 2 < NCHT)
                def _():
                    gath(i + NBUF - 2)

        @pl.loop(NCHT - 2, NCHT)
        def _(i):
            wait_scat(i)

        plsc.subcore_barrier()
        pltpu.sync_copy(
            acc.at[pl.ds(sid * RPT, RPT)], s.at[pl.ds(sid * RPT, RPT)]
        )

    for r, (ga, gb, sa, sb) in enumerate(((g0, g1, s0, s1),
                                          (g2, g3, s2, s3))):
        @pl.when(cid == 0)
        def _(ga=ga, sa=sa):
            run(ga, sa)

        @pl.when(cid == 1)
        def _(gb=gb, sb=sb):
            run(gb, sb)


_seg_call = pl.kernel(
    _seg_body,
    out_type=(jax.ShapeDtypeStruct((NP, DQ), jnp.float32),) * 4,
    mesh=_mesh,
    scratch_types=[
        pltpu.VMEM((NCHT, ECH), jnp.int32),
        pltpu.VMEM((NCHT, ECH), jnp.int32),
        pltpu.VMEM((NBUF, ECH, DQ), jnp.float32),
        pltpu.VMEM_SHARED((NP, DQ), jnp.float32),
        pltpu.SemaphoreType.DMA,
        pltpu.SemaphoreType.DMA,
    ],
    compiler_params=pltpu.CompilerParams(use_tc_tiling_on_sc=False),
)


def _dinv_of(degp_ref):
    deg = 1.0 + degp_ref[0][:, 0:1] + degp_ref[1][:, 0:1]
    return lax.rsqrt(deg)


def _tc1_body(x_ref, w_ref, degp_ref, *g_refs):
    dinv = _dinv_of(degp_ref)
    h = jnp.dot(x_ref[...], w_ref[...], preferred_element_type=jnp.float32)
    g = h * dinv
    for q, g_ref in enumerate(g_refs):
        g_ref[...] = g[:, q * DQ:(q + 1) * DQ]


def _tc2_body(s0, s1, s2, s3, g0, g1, g2, g3, degp_ref, b_ref, w_ref,
              *g2_refs):
    dinv = _dinv_of(degp_ref)
    s = jnp.concatenate([s0[...], s1[...], s2[...], s3[...]], axis=1)
    g = jnp.concatenate([g0[...], g1[...], g2[...], g3[...]], axis=1)
    a = jnp.maximum((s + g) * dinv + b_ref[...], 0.0)
    h = jnp.dot(a, w_ref[...], preferred_element_type=jnp.float32)
    gg = h * dinv
    for q, g_ref in enumerate(g2_refs):
        g_ref[...] = gg[:, q * DQ:(q + 1) * DQ]


def _tc3_body(s0, s1, s2, s3, g0, g1, g2, g3, degp_ref, b_ref, x_ref,
              out_ref):
    dinv = _dinv_of(degp_ref)
    s = jnp.concatenate([s0[...], s1[...], s2[...], s3[...]], axis=1)
    g = jnp.concatenate([g0[...], g1[...], g2[...], g3[...]], axis=1)
    out_ref[...] = (s + g) * dinv + b_ref[...] + x_ref[...]


_q_spec = pl.BlockSpec((BN, DQ), lambda i: (i, 0))
_full_spec = pl.BlockSpec((BN, D), lambda i: (i, 0))
_w_spec = pl.BlockSpec((D, D), lambda i: (0, 0))
_degp_spec = pl.BlockSpec((2, BN, 16), lambda i: (0, i, 0))
_b_spec = pl.BlockSpec((1, D), lambda i: (0, 0))

_tc1_call = pl.pallas_call(
    _tc1_body,
    grid=(NP // BN,),
    in_specs=[_full_spec, _w_spec, _degp_spec],
    out_specs=[_q_spec] * 4,
    out_shape=[jax.ShapeDtypeStruct((NP, DQ), jnp.float32)] * 4,
)

_tc2_call = pl.pallas_call(
    _tc2_body,
    grid=(NP // BN,),
    in_specs=[_q_spec] * 8 + [_degp_spec, _b_spec, _w_spec],
    out_specs=[_q_spec] * 4,
    out_shape=[jax.ShapeDtypeStruct((NP, DQ), jnp.float32)] * 4,
)

_tc3_call = pl.pallas_call(
    _tc3_body,
    grid=(NP // BN,),
    in_specs=[_q_spec] * 8 + [_degp_spec, _b_spec, _full_spec],
    out_specs=_full_spec,
    out_shape=jax.ShapeDtypeStruct((NP, D), jnp.float32),
)


def kernel(x, edge_index, W1, b1, W2, b2):
    src = edge_index[0]
    dst = edge_index[1]
    pad = jnp.full((EP - E,), DUMP, dtype=jnp.int32)
    src2d = jnp.concatenate([src, pad]).reshape(EP // ECH, ECH)
    dst2d = jnp.concatenate([dst, pad]).reshape(EP // ECH, ECH)
    xp = jnp.zeros((NP, D), jnp.float32).at[:N].set(x)
    w1t = W1.T
    w2t = W2.T
    zrows = jnp.zeros((RPT, DQ), jnp.float32)
    z16 = jnp.zeros((RPT, 16), jnp.float32)
    ones16 = jnp.zeros((ECH, 16), jnp.float32).at[:, 0].set(1.0)

    degp = _deg_call(dst2d, ones16, z16)
    gs = _tc1_call(xp, w1t, degp)
    ss = _seg_call(*gs, src2d, dst2d, zrows)
    g2s = _tc2_call(*ss, *gs, degp, b1.reshape(1, D), w2t)
    s2s = _seg_call(*g2s, src2d, dst2d, zrows)
    out = _tc3_call(*s2s, *g2s, degp, b2.reshape(1, D), xp)
    return out[:N]


# speedup vs baseline: 11.1958x; 1.4940x over previous
import jax
import jax.numpy as jnp
from jax import lax
from jax.experimental import pallas as pl
from jax.experimental.pallas import tpu as pltpu
from jax.experimental.pallas import tpu_sc as plsc

N = 10000
E = 160000
D = 256
DH = 128
NP = 10240
ECH = 128
EP = 163840
DUMP = N
NSUB = 16
RPT = NP // NSUB
NCHT = EP // ECH // NSUB
NCHD = EP // ECH // (2 * NSUB)
NBUF = 6
BN = 256

_mesh = plsc.VectorSubcoreMesh(
    core_axis_name="c", subcore_axis_name="s", num_cores=2, num_subcores=NSUB
)


def _deg_body(dst2d, ones16, z16, degp, dst_buf, ones_v, acc):
    cid = lax.axis_index("c")
    sid = lax.axis_index("s")
    wid = cid * NSUB + sid
    pltpu.sync_copy(z16, acc.at[pl.ds(sid * RPT, RPT)])
    pltpu.sync_copy(ones16, ones_v)
    pltpu.sync_copy(dst2d.at[pl.ds(wid * NCHD, NCHD)], dst_buf)
    plsc.subcore_barrier()

    @pl.loop(0, NCHD)
    def _(i):
        pltpu.sync_copy(ones_v, acc.at[dst_buf.at[i]], add=True)

    plsc.subcore_barrier()
    pltpu.sync_copy(
        acc.at[pl.ds(sid * RPT, RPT)], degp.at[cid, pl.ds(sid * RPT, RPT)]
    )


_deg_call = pl.kernel(
    _deg_body,
    out_type=jax.ShapeDtypeStruct((2, NP, 16), jnp.float32),
    mesh=_mesh,
    scratch_types=[
        pltpu.VMEM((NCHD, ECH), jnp.int32),
        pltpu.VMEM((ECH, 16), jnp.float32),
        pltpu.VMEM_SHARED((NP, 16), jnp.float32),
    ],
    compiler_params=pltpu.CompilerParams(use_tc_tiling_on_sc=False),
)


def _seg_body(g0, g1, src2d, dst2d, zrows, s0, s1,
              src_buf, dst_buf, data, acc, gsem, ssem):
    cid = lax.axis_index("c")
    sid = lax.axis_index("s")
    base = sid * NCHT
    pltpu.sync_copy(src2d.at[pl.ds(base, NCHT)], src_buf)
    pltpu.sync_copy(dst2d.at[pl.ds(base, NCHT)], dst_buf)

    def run(g, s):
        pltpu.sync_copy(zrows, acc.at[pl.ds(sid * RPT, RPT)])
        plsc.subcore_barrier()

        def gath(i):
            pltpu.async_copy(
                g.at[src_buf.at[i]], data.at[lax.rem(i, NBUF)], gsem
            )

        def wait_gath(i):
            pltpu.make_async_copy(
                g.at[src_buf.at[i]], data.at[lax.rem(i, NBUF)], gsem
            ).wait()

        def scat(i):
            pltpu.async_copy(
                data.at[lax.rem(i, NBUF)], acc.at[dst_buf.at[i]], ssem,
                add=True,
            )

        def wait_scat(i):
            pltpu.make_async_copy(
                data.at[lax.rem(i, NBUF)], acc.at[dst_buf.at[i]], ssem
            ).wait()

        for b in range(NBUF):
            gath(b)

        @pl.loop(0, NCHT)
        def _(i):
            wait_gath(i)
            scat(i)

            @pl.when(i >= 2)
            def _():
                wait_scat(i - 2)

                @pl.when(i + NBUF - 2 < NCHT)
                def _():
                    gath(i + NBUF - 2)

        @pl.loop(NCHT - 2, NCHT)
        def _(i):
            wait_scat(i)

        plsc.subcore_barrier()
        pltpu.sync_copy(
            acc.at[pl.ds(sid * RPT, RPT)], s.at[pl.ds(sid * RPT, RPT)]
        )

    @pl.when(cid == 0)
    def _():
        run(g0, s0)

    @pl.when(cid == 1)
    def _():
        run(g1, s1)


_seg_call = pl.kernel(
    _seg_body,
    out_type=(jax.ShapeDtypeStruct((NP, DH), jnp.bfloat16),) * 2,
    mesh=_mesh,
    scratch_types=[
        pltpu.VMEM((NCHT, ECH), jnp.int32),
        pltpu.VMEM((NCHT, ECH), jnp.int32),
        pltpu.VMEM((NBUF, ECH, DH), jnp.bfloat16),
        pltpu.VMEM_SHARED((NP, DH), jnp.bfloat16),
        pltpu.SemaphoreType.DMA,
        pltpu.SemaphoreType.DMA,
    ],
    compiler_params=pltpu.CompilerParams(use_tc_tiling_on_sc=False),
)


def _dinv_of(degp_ref):
    deg = 1.0 + degp_ref[0][:, 0:1] + degp_ref[1][:, 0:1]
    return lax.rsqrt(deg)


def _tc1_body(x_ref, w_ref, degp_ref, g0_ref, g1_ref):
    dinv = _dinv_of(degp_ref)
    h = jnp.dot(x_ref[...], w_ref[...], preferred_element_type=jnp.float32)
    g = (h * dinv).astype(jnp.bfloat16)
    g0_ref[...] = g[:, :DH]
    g1_ref[...] = g[:, DH:]


def _tc2_body(s0, s1, g0, g1, degp_ref, b_ref, w_ref, g20_ref, g21_ref):
    dinv = _dinv_of(degp_ref)
    s = jnp.concatenate([s0[...], s1[...]], axis=1).astype(jnp.float32)
    g = jnp.concatenate([g0[...], g1[...]], axis=1).astype(jnp.float32)
    a = jnp.maximum((s + g) * dinv + b_ref[...], 0.0)
    h = jnp.dot(a, w_ref[...], preferred_element_type=jnp.float32)
    gg = (h * dinv).astype(jnp.bfloat16)
    g20_ref[...] = gg[:, :DH]
    g21_ref[...] = gg[:, DH:]


def _tc3_body(s0, s1, g0, g1, degp_ref, b_ref, x_ref, out_ref):
    dinv = _dinv_of(degp_ref)
    s = jnp.concatenate([s0[...], s1[...]], axis=1).astype(jnp.float32)
    g = jnp.concatenate([g0[...], g1[...]], axis=1).astype(jnp.float32)
    out_ref[...] = (s + g) * dinv + b_ref[...] + x_ref[...]


_h_spec = pl.BlockSpec((BN, DH), lambda i: (i, 0))
_full_spec = pl.BlockSpec((BN, D), lambda i: (i, 0))
_w_spec = pl.BlockSpec((D, D), lambda i: (0, 0))
_degp_spec = pl.BlockSpec((2, BN, 16), lambda i: (0, i, 0))
_b_spec = pl.BlockSpec((1, D), lambda i: (0, 0))

_tc1_call = pl.pallas_call(
    _tc1_body,
    grid=(NP // BN,),
    in_specs=[_full_spec, _w_spec, _degp_spec],
    out_specs=[_h_spec] * 2,
    out_shape=[jax.ShapeDtypeStruct((NP, DH), jnp.bfloat16)] * 2,
)

_tc2_call = pl.pallas_call(
    _tc2_body,
    grid=(NP // BN,),
    in_specs=[_h_spec] * 4 + [_degp_spec, _b_spec, _w_spec],
    out_specs=[_h_spec] * 2,
    out_shape=[jax.ShapeDtypeStruct((NP, DH), jnp.bfloat16)] * 2,
)

_tc3_call = pl.pallas_call(
    _tc3_body,
    grid=(NP // BN,),
    in_specs=[_h_spec] * 4 + [_degp_spec, _b_spec, _full_spec],
    out_specs=_full_spec,
    out_shape=jax.ShapeDtypeStruct((NP, D), jnp.float32),
)


def kernel(x, edge_index, W1, b1, W2, b2):
    src = edge_index[0]
    dst = edge_index[1]
    pad = jnp.full((EP - E,), DUMP, dtype=jnp.int32)
    src2d = jnp.concatenate([src, pad]).reshape(EP // ECH, ECH)
    dst2d = jnp.concatenate([dst, pad]).reshape(EP // ECH, ECH)
    xp = jnp.zeros((NP, D), jnp.float32).at[:N].set(x)
    w1t = W1.T
    w2t = W2.T
    zrows = jnp.zeros((RPT, DH), jnp.bfloat16)
    z16 = jnp.zeros((RPT, 16), jnp.float32)
    ones16 = jnp.zeros((ECH, 16), jnp.float32).at[:, 0].set(1.0)

    degp = _deg_call(dst2d, ones16, z16)
    gs = _tc1_call(xp, w1t, degp)
    ss = _seg_call(*gs, src2d, dst2d, zrows)
    g2s = _tc2_call(*ss, *gs, degp, b1.reshape(1, D), w2t)
    s2s = _seg_call(*g2s, src2d, dst2d, zrows)
    out = _tc3_call(*s2s, *g2s, degp, b2.reshape(1, D), xp)
    return out[:N]


# speedup vs baseline: 12.1248x; 1.0830x over previous
import jax
import jax.numpy as jnp
from jax import lax
from jax.experimental import pallas as pl
from jax.experimental.pallas import tpu as pltpu
from jax.experimental.pallas import tpu_sc as plsc

N = 10000
E = 160000
D = 256
DH = 128
NP = 10240
ECH = 128
EP = 163840
DPAD = N
SPAD = 0
NSUB = 16
RPT = NP // NSUB
NCHT = EP // ECH // NSUB
NCHD = EP // ECH // (2 * NSUB)
NBUF = 6
BN = 400

_mesh = plsc.VectorSubcoreMesh(
    core_axis_name="c", subcore_axis_name="s", num_cores=2, num_subcores=NSUB
)


def _deg_body(dst2d, ones16, z16, degp, dst_buf, ones_v, acc):
    cid = lax.axis_index("c")
    sid = lax.axis_index("s")
    wid = cid * NSUB + sid
    pltpu.sync_copy(z16, acc.at[pl.ds(sid * RPT, RPT)])
    pltpu.sync_copy(ones16, ones_v)
    pltpu.sync_copy(dst2d.at[pl.ds(wid * NCHD, NCHD)], dst_buf)
    plsc.subcore_barrier()

    @pl.loop(0, NCHD)
    def _(i):
        pltpu.sync_copy(ones_v, acc.at[dst_buf.at[i]], add=True)

    plsc.subcore_barrier()
    pltpu.sync_copy(
        acc.at[pl.ds(sid * RPT, RPT)], degp.at[cid, pl.ds(sid * RPT, RPT)]
    )


_deg_call = pl.kernel(
    _deg_body,
    out_type=jax.ShapeDtypeStruct((2, NP, 16), jnp.float32),
    mesh=_mesh,
    scratch_types=[
        pltpu.VMEM((NCHD, ECH), jnp.int32),
        pltpu.VMEM((ECH, 16), jnp.float32),
        pltpu.VMEM_SHARED((NP, 16), jnp.float32),
    ],
    compiler_params=pltpu.CompilerParams(use_tc_tiling_on_sc=False),
)


def _seg_body(g0, g1, src2d, dst2d, zrows, s0, s1,
              src_buf, dst_buf, data, acc, gsem, ssem):
    cid = lax.axis_index("c")
    sid = lax.axis_index("s")
    base = sid * NCHT
    pltpu.sync_copy(src2d.at[pl.ds(base, NCHT)], src_buf)
    pltpu.sync_copy(dst2d.at[pl.ds(base, NCHT)], dst_buf)

    def run(g, s):
        pltpu.sync_copy(zrows, acc.at[pl.ds(sid * RPT, RPT)])
        plsc.subcore_barrier()

        def gath(i):
            pltpu.async_copy(
                g.at[src_buf.at[i]], data.at[lax.rem(i, NBUF)], gsem
            )

        def wait_gath(i):
            pltpu.make_async_copy(
                g.at[src_buf.at[i]], data.at[lax.rem(i, NBUF)], gsem
            ).wait()

        def scat(i):
            pltpu.async_copy(
                data.at[lax.rem(i, NBUF)], acc.at[dst_buf.at[i]], ssem,
                add=True,
            )

        def wait_scat(i):
            pltpu.make_async_copy(
                data.at[lax.rem(i, NBUF)], acc.at[dst_buf.at[i]], ssem
            ).wait()

        for b in range(NBUF):
            gath(b)

        @pl.loop(0, NCHT)
        def _(i):
            wait_gath(i)
            scat(i)

            @pl.when(i >= 2)
            def _():
                wait_scat(i - 2)

                @pl.when(i + NBUF - 2 < NCHT)
                def _():
                    gath(i + NBUF - 2)

        @pl.loop(NCHT - 2, NCHT)
        def _(i):
            wait_scat(i)

        plsc.subcore_barrier()
        pltpu.sync_copy(
            acc.at[pl.ds(sid * RPT, RPT)], s.at[pl.ds(sid * RPT, RPT)]
        )

    @pl.when(cid == 0)
    def _():
        run(g0, s0)

    @pl.when(cid == 1)
    def _():
        run(g1, s1)


_seg_call = pl.kernel(
    _seg_body,
    out_type=(jax.ShapeDtypeStruct((NP, DH), jnp.bfloat16),) * 2,
    mesh=_mesh,
    scratch_types=[
        pltpu.VMEM((NCHT, ECH), jnp.int32),
        pltpu.VMEM((NCHT, ECH), jnp.int32),
        pltpu.VMEM((NBUF, ECH, DH), jnp.bfloat16),
        pltpu.VMEM_SHARED((NP, DH), jnp.bfloat16),
        pltpu.SemaphoreType.DMA,
        pltpu.SemaphoreType.DMA,
    ],
    compiler_params=pltpu.CompilerParams(use_tc_tiling_on_sc=False),
)


def _dinv_of(degp_ref):
    deg = 1.0 + degp_ref[0][:, 0:1] + degp_ref[1][:, 0:1]
    return lax.rsqrt(deg)


def _tc1_body(x_ref, w_ref, degp_ref, g0_ref, g1_ref):
    dinv = _dinv_of(degp_ref)
    h = jnp.dot(x_ref[...], w_ref[...], preferred_element_type=jnp.float32)
    g = (h * dinv).astype(jnp.bfloat16)
    g0_ref[...] = g[:, :DH]
    g1_ref[...] = g[:, DH:]


def _tc2_body(s0, s1, g0, g1, degp_ref, b_ref, w_ref, g20_ref, g21_ref):
    dinv = _dinv_of(degp_ref)
    s = jnp.concatenate([s0[...], s1[...]], axis=1).astype(jnp.float32)
    g = jnp.concatenate([g0[...], g1[...]], axis=1).astype(jnp.float32)
    a = jnp.maximum((s + g) * dinv + b_ref[...], 0.0)
    h = jnp.dot(a, w_ref[...], preferred_element_type=jnp.float32)
    gg = (h * dinv).astype(jnp.bfloat16)
    g20_ref[...] = gg[:, :DH]
    g21_ref[...] = gg[:, DH:]


def _tc3_body(s0, s1, g0, g1, degp_ref, b_ref, x_ref, out_ref):
    dinv = _dinv_of(degp_ref)
    s = jnp.concatenate([s0[...], s1[...]], axis=1).astype(jnp.float32)
    g = jnp.concatenate([g0[...], g1[...]], axis=1).astype(jnp.float32)
    out_ref[...] = (s + g) * dinv + b_ref[...] + x_ref[...]


_h_spec = pl.BlockSpec((BN, DH), lambda i: (i, 0))
_full_spec = pl.BlockSpec((BN, D), lambda i: (i, 0))
_w_spec = pl.BlockSpec((D, D), lambda i: (0, 0))
_degp_spec = pl.BlockSpec((2, BN, 16), lambda i: (0, i, 0))
_b_spec = pl.BlockSpec((1, D), lambda i: (0, 0))

_tc1_call = pl.pallas_call(
    _tc1_body,
    grid=(N // BN,),
    in_specs=[_full_spec, _w_spec, _degp_spec],
    out_specs=[_h_spec] * 2,
    out_shape=[jax.ShapeDtypeStruct((N, DH), jnp.bfloat16)] * 2,
)

_tc2_call = pl.pallas_call(
    _tc2_body,
    grid=(N // BN,),
    in_specs=[_h_spec] * 4 + [_degp_spec, _b_spec, _w_spec],
    out_specs=[_h_spec] * 2,
    out_shape=[jax.ShapeDtypeStruct((N, DH), jnp.bfloat16)] * 2,
)

_tc3_call = pl.pallas_call(
    _tc3_body,
    grid=(N // BN,),
    in_specs=[_h_spec] * 4 + [_degp_spec, _b_spec, _full_spec],
    out_specs=_full_spec,
    out_shape=jax.ShapeDtypeStruct((N, D), jnp.float32),
)


def kernel(x, edge_index, W1, b1, W2, b2):
    src = edge_index[0]
    dst = edge_index[1]
    spad = jnp.full((EP - E,), SPAD, dtype=jnp.int32)
    dpad = jnp.full((EP - E,), DPAD, dtype=jnp.int32)
    src2d = jnp.concatenate([src, spad]).reshape(EP // ECH, ECH)
    dst2d = jnp.concatenate([dst, dpad]).reshape(EP // ECH, ECH)
    w1t = W1.T
    w2t = W2.T
    zrows = jnp.zeros((RPT, DH), jnp.bfloat16)
    z16 = jnp.zeros((RPT, 16), jnp.float32)
    ones16 = jnp.zeros((ECH, 16), jnp.float32).at[:, 0].set(1.0)

    degp = _deg_call(dst2d, ones16, z16)
    gs = _tc1_call(x, w1t, degp)
    ss = _seg_call(*gs, src2d, dst2d, zrows)
    g2s = _tc2_call(*ss, *gs, degp, b1.reshape(1, D), w2t)
    s2s = _seg_call(*g2s, src2d, dst2d, zrows)
    return _tc3_call(*s2s, *g2s, degp, b2.reshape(1, D), x)
